# Initial kernel scaffold; baseline (speedup 1.0000x reference)
import jax
import jax.numpy as jnp
from jax.experimental import pallas as pl


def kernel(atom_feat, bond_feat, angle_feat, atom_edge_index, bond_edge_index, angle_index, atom_bond_weight, bond_node_weight, params):
    raise NotImplementedError("write your pallas kernel here")


# speedup vs baseline: 1.3808x; 1.3808x over previous
import functools

import jax
import jax.numpy as jnp
from jax import lax
from jax.experimental import pallas as pl
from jax.experimental.pallas import tpu as pltpu
from jax.experimental.pallas import tpu_sc as plsc

_NA = 10000
_EA = 160000
_EB = 320000
_D = 128
_H = 256

_NC = 2
_NS = 16
_NW = _NC * _NS
_CK = 128

_SP1 = 10240
_SP2C = 8192
_NCH2 = 20
_SP2 = _SP2C * _NCH2


def _mesh():
    return plsc.VectorSubcoreMesh(core_axis_name="c", subcore_axis_name="s")


def _zero_zb(zb):
    def zrow(i, carry):
        for v in range(_D // 16):
            zb[i, pl.ds(v * 16, 16)] = jnp.zeros((16,), jnp.float32)
        return carry

    lax.fori_loop(0, _CK, zrow, 0)


def _gather2_body(nch, table, idx0, idx1, out0, out1,
                  iv0, iv1, buf0, buf1, sem0, sem1):
    wid = lax.axis_index("s") * _NC + lax.axis_index("c")
    nj = (nch + _NW - 1) // _NW

    def step(j, carry):
        c = wid + j * _NW

        @pl.when(c < nch)
        def _():
            r0 = c * _CK
            pltpu.sync_copy(idx0.at[pl.ds(r0, _CK)], iv0)
            pltpu.sync_copy(idx1.at[pl.ds(r0, _CK)], iv1)
            cp0 = pltpu.async_copy(table.at[iv0], buf0, sem0)
            cp1 = pltpu.async_copy(table.at[iv1], buf1, sem1)
            cp0.wait()
            cp1.wait()
            pltpu.sync_copy(buf0, out0.at[pl.ds(r0, _CK)])
            pltpu.sync_copy(buf1, out1.at[pl.ds(r0, _CK)])

        return carry

    lax.fori_loop(0, nj, step, 0)


def _gather2(table, idx0, idx1):
    e = idx0.shape[0]
    nch = e // _CK
    out = jax.ShapeDtypeStruct((e, _D), jnp.float32)
    f = pl.kernel(
        functools.partial(_gather2_body, nch),
        out_type=(out, out),
        mesh=_mesh(),
        scratch_types=[
            pltpu.VMEM((_CK,), jnp.int32),
            pltpu.VMEM((_CK,), jnp.int32),
            pltpu.VMEM((_CK, _D), jnp.float32),
            pltpu.VMEM((_CK, _D), jnp.float32),
            pltpu.SemaphoreType.DMA,
            pltpu.SemaphoreType.DMA,
        ],
    )
    return f(table, idx0, idx1)


def _gather5_body(nch, bond, bnw, atom, idx_s, idx_d, idx_v,
                  o_bs, o_bd, o_ws, o_wd, o_vf,
                  ivs, ivd, ivv, b0, b1, b2, b3, b4,
                  s0, s1, s2, s3, s4):
    wid = lax.axis_index("s") * _NC + lax.axis_index("c")
    nj = (nch + _NW - 1) // _NW

    def step(j, carry):
        c = wid + j * _NW

        @pl.when(c < nch)
        def _():
            r0 = c * _CK
            pltpu.sync_copy(idx_s.at[pl.ds(r0, _CK)], ivs)
            pltpu.sync_copy(idx_d.at[pl.ds(r0, _CK)], ivd)
            pltpu.sync_copy(idx_v.at[pl.ds(r0, _CK)], ivv)
            c0 = pltpu.async_copy(bond.at[ivs], b0, s0)
            c1 = pltpu.async_copy(bond.at[ivd], b1, s1)
            c2 = pltpu.async_copy(bnw.at[ivs], b2, s2)
            c3 = pltpu.async_copy(bnw.at[ivd], b3, s3)
            c4 = pltpu.async_copy(atom.at[ivv], b4, s4)
            c0.wait()
            c1.wait()
            c2.wait()
            c3.wait()
            c4.wait()
            pltpu.sync_copy(b0, o_bs.at[pl.ds(r0, _CK)])
            pltpu.sync_copy(b1, o_bd.at[pl.ds(r0, _CK)])
            pltpu.sync_copy(b2, o_ws.at[pl.ds(r0, _CK)])
            pltpu.sync_copy(b3, o_wd.at[pl.ds(r0, _CK)])
            pltpu.sync_copy(b4, o_vf.at[pl.ds(r0, _CK)])

        return carry

    lax.fori_loop(0, nj, step, 0)


def _gather5(bond, bnw, atom, idx_s, idx_d, idx_v):
    e = idx_s.shape[0]
    nch = e // _CK
    out = jax.ShapeDtypeStruct((e, _D), jnp.float32)
    f = pl.kernel(
        functools.partial(_gather5_body, nch),
        out_type=(out,) * 5,
        mesh=_mesh(),
        scratch_types=[
            pltpu.VMEM((_CK,), jnp.int32),
            pltpu.VMEM((_CK,), jnp.int32),
            pltpu.VMEM((_CK,), jnp.int32),
        ] + [pltpu.VMEM((_CK, _D), jnp.float32)] * 5
          + [pltpu.SemaphoreType.DMA] * 5,
    )
    return f(bond, bnw, atom, idx_s, idx_d, idx_v)


def _segsum1_body(m, dstv, hout, acc, buf, lidx, zb):
    cid = lax.axis_index("c")
    sid = lax.axis_index("s")
    _zero_zb(zb)
    rpt = _SP1 // _NS
    for r in range(rpt // _CK):
        pltpu.sync_copy(zb, acc.at[pl.ds(sid * rpt + r * _CK, _CK)])
    plsc.subcore_barrier()

    nch = _EA // _CK
    nch_core = nch // _NC
    nj = (nch_core + _NS - 1) // _NS

    def step(j, carry):
        lc = sid + j * _NS

        @pl.when(lc < nch_core)
        def _():
            r0 = (cid * nch_core + lc) * _CK
            pltpu.sync_copy(dstv.at[pl.ds(r0, _CK)], lidx)
            pltpu.sync_copy(m.at[pl.ds(r0, _CK)], buf)
            pltpu.sync_copy(buf, acc.at[lidx], add=True)

        return carry

    lax.fori_loop(0, nj, step, 0)
    plsc.subcore_barrier()
    for r in range(rpt // _CK):
        rr = sid * rpt + r * _CK
        pltpu.sync_copy(acc.at[pl.ds(rr, _CK)], buf)
        pltpu.sync_copy(buf, hout.at[cid, pl.ds(rr, _CK)])


def _segsum1(m, dstv):
    f = pl.kernel(
        _segsum1_body,
        out_type=jax.ShapeDtypeStruct((_NC, _SP1, _D), jnp.float32),
        mesh=_mesh(),
        scratch_types=[
            pltpu.VMEM_SHARED((_SP1, _D), jnp.float32),
            pltpu.VMEM((_CK, _D), jnp.float32),
            pltpu.VMEM((_CK,), jnp.int32),
            pltpu.VMEM((_CK, _D), jnp.float32),
        ],
    )
    return f(m, dstv)


def _segsum2_body(m, dstv, hout, acc, buf, dbuf, lidx, zb):
    cid = lax.axis_index("c")
    sid = lax.axis_index("s")
    _zero_zb(zb)
    nch = _EB // _CK
    nje = (nch + _NS - 1) // _NS
    rpt = _SP2C // _NS

    def chunk_loop(k, carry):
        lo = (cid + k * _NC) * _SP2C
        for r in range(rpt // _CK):
            pltpu.sync_copy(zb, acc.at[pl.ds(sid * rpt + r * _CK, _CK)])
        plsc.subcore_barrier()

        def estep(j, ecarry):
            ec = sid + j * _NS

            @pl.when(ec < nch)
            def _():
                r0 = ec * _CK
                pltpu.sync_copy(dstv.at[pl.ds(r0, _CK)], dbuf)
                pltpu.sync_copy(m.at[pl.ds(r0, _CK)], buf)
                for v in range(_CK // 16):
                    dd = dbuf[pl.ds(v * 16, 16)]
                    rel = dd - lo
                    ok = (rel >= 0) & (rel < _SP2C)
                    lidx[pl.ds(v * 16, 16)] = jnp.where(ok, rel, _SP2C)
                pltpu.sync_copy(buf, acc.at[lidx], add=True)

            return ecarry

        lax.fori_loop(0, nje, estep, 0)
        plsc.subcore_barrier()
        for r in range(rpt // _CK):
            rr = sid * rpt + r * _CK
            pltpu.sync_copy(acc.at[pl.ds(rr, _CK)], buf)
            pltpu.sync_copy(buf, hout.at[pl.ds(lo + rr, _CK)])
        plsc.subcore_barrier()
        return carry

    lax.fori_loop(0, _NCH2 // _NC, chunk_loop, 0)


def _segsum2(m, dstv):
    f = pl.kernel(
        _segsum2_body,
        out_type=jax.ShapeDtypeStruct((_SP2, _D), jnp.float32),
        mesh=_mesh(),
        scratch_types=[
            pltpu.VMEM_SHARED((_SP2C + 8, _D), jnp.float32),
            pltpu.VMEM((_CK, _D), jnp.float32),
            pltpu.VMEM((_CK,), jnp.int32),
            pltpu.VMEM((_CK,), jnp.int32),
            pltpu.VMEM((_CK, _D), jnp.float32),
        ],
    )
    return f(m, dstv)


_BE = 1280


def _sigmoid(x):
    return 1.0 / (1.0 + jnp.exp(-x))


def _silu(x):
    return x * _sigmoid(x)


def _mlp_body(n_in, n_mult, residual, *refs):
    xs = refs[:n_in]
    i = n_in
    mults = refs[i:i + n_mult]
    i += n_mult
    res = refs[i] if residual else None
    i += 1 if residual else 0
    gw1, gb1, gw2, gb2, ow1, ob1, ow2, ob2, out = refs[i:i + 9]

    x0 = xs[0][...]
    ag = jnp.dot(x0, gw1[0], preferred_element_type=jnp.float32)
    ao = jnp.dot(x0, ow1[0], preferred_element_type=jnp.float32)
    for k in range(1, n_in):
        xk = xs[k][...]
        ag += jnp.dot(xk, gw1[k], preferred_element_type=jnp.float32)
        ao += jnp.dot(xk, ow1[k], preferred_element_type=jnp.float32)
    hg = _silu(ag + gb1[...])
    ho = _silu(ao + ob1[...])
    g = _sigmoid(jnp.dot(hg, gw2[...], preferred_element_type=jnp.float32)
                 + gb2[...])
    o = _silu(jnp.dot(ho, ow2[...], preferred_element_type=jnp.float32)
              + ob2[...])
    y = o * g
    for mr in mults:
        y = y * mr[...]
    if residual:
        y = y + res[...]
    out[...] = y


def _mlp(xs, mults, res, p):
    e = xs[0].shape[0]
    n_in = len(xs)
    grid = (e // _BE,)
    row = pl.BlockSpec((_BE, _D), lambda i: (i, 0))
    w1s = pl.BlockSpec((n_in, _D, _H), lambda i: (0, 0, 0))
    b1s = pl.BlockSpec((1, _H), lambda i: (0, 0))
    w2s = pl.BlockSpec((_H, _D), lambda i: (0, 0))
    b2s = pl.BlockSpec((1, _D), lambda i: (0, 0))
    n_row = n_in + len(mults) + (1 if res is not None else 0)
    in_specs = [row] * n_row + [w1s, b1s, w2s, b2s, w1s, b1s, w2s, b2s]
    gw1 = p['gw1'].reshape(n_in, _D, _H)
    ow1 = p['ow1'].reshape(n_in, _D, _H)
    args = ([*xs, *mults] + ([res] if res is not None else [])
            + [gw1, p['gb1'].reshape(1, _H), p['gw2'], p['gb2'].reshape(1, _D),
               ow1, p['ob1'].reshape(1, _H), p['ow2'], p['ob2'].reshape(1, _D)])
    return pl.pallas_call(
        functools.partial(_mlp_body, n_in, len(mults), res is not None),
        grid=grid,
        in_specs=in_specs,
        out_specs=row,
        out_shape=jax.ShapeDtypeStruct((e, _D), jnp.float32),
        compiler_params=pltpu.CompilerParams(
            dimension_semantics=("arbitrary",)),
    )(*args)


def _lin_body(nh, *refs):
    feat = refs[0]
    hs = refs[1:1 + nh]
    w, b, out = refs[1 + nh:1 + nh + 3]
    h = hs[0][...]
    for k in range(1, nh):
        h = h + hs[k][...]
    out[...] = (feat[...]
                + jnp.dot(h, w[...], preferred_element_type=jnp.float32)
                + b[...])


def _lin(feat, hs, w, b, be):
    e = feat.shape[0]
    grid = (e // be,)
    row = pl.BlockSpec((be, _D), lambda i: (i, 0))
    ws = pl.BlockSpec((_D, _D), lambda i: (0, 0))
    bs = pl.BlockSpec((1, _D), lambda i: (0, 0))
    return pl.pallas_call(
        functools.partial(_lin_body, len(hs)),
        grid=grid,
        in_specs=[row] * (1 + len(hs)) + [ws, bs],
        out_specs=row,
        out_shape=jax.ShapeDtypeStruct((e, _D), jnp.float32),
        compiler_params=pltpu.CompilerParams(
            dimension_semantics=("arbitrary",)),
    )(feat, *hs, w, b.reshape(1, _D))


def kernel(atom_feat, bond_feat, angle_feat, atom_edge_index, bond_edge_index,
           angle_index, atom_bond_weight, bond_node_weight, params):
    src_a = atom_edge_index[0]
    dst_a = atom_edge_index[1]
    src_b = bond_edge_index[0]
    dst_b = bond_edge_index[1]
    vertex = angle_index[:, 1]

    g1s, g1d = _gather2(atom_feat, src_a, dst_a)
    m1 = _mlp([g1s, g1d, bond_feat], [atom_bond_weight], None,
              params['atom_conv'])
    hparts = _segsum1(m1, dst_a)
    atom_out = _lin(atom_feat, [hparts[0, :_NA], hparts[1, :_NA]],
                    params['atom_lin']['w'], params['atom_lin']['b'], 1000)

    bfs, bfd, ws, wd, vf = _gather5(bond_feat, bond_node_weight, atom_out,
                                    src_b, dst_b, vertex)
    m2 = _mlp([bfs, bfd, angle_feat, vf], [ws, wd], None, params['bond_conv'])
    h2 = _segsum2(m2, dst_b)
    bond_out = _lin(bond_feat, [h2[:_EA]],
                    params['bond_lin']['w'], params['bond_lin']['b'], _BE)

    g3s, g3d = _gather2(bond_out, src_b, dst_b)
    angle_out = _mlp([g3s, g3d, angle_feat, vf], [], angle_feat,
                     params['angle_update'])

    return (atom_out, bond_out, angle_out)


# speedup vs baseline: 1.5330x; 1.1102x over previous
import functools

import jax
import jax.numpy as jnp
from jax import lax
from jax.experimental import pallas as pl
from jax.experimental.pallas import tpu as pltpu
from jax.experimental.pallas import tpu_sc as plsc

_NA = 10000
_EA = 160000
_EB = 320000
_D = 128
_H = 256

_NC = 2
_NS = 16
_NW = _NC * _NS
_CK = 128

_SP1 = 10240
_SP2C = 8192
_NCH2 = 20
_SP2 = _SP2C * _NCH2


def _mesh():
    return plsc.VectorSubcoreMesh(core_axis_name="c", subcore_axis_name="s")


def _zero_zb(zb):
    def zrow(i, carry):
        for v in range(_D // 16):
            zb[i, pl.ds(v * 16, 16)] = jnp.zeros((16,), jnp.float32)
        return carry

    lax.fori_loop(0, _CK, zrow, 0)


def _gather2_body(nch, table, idx0, idx1, out0, out1,
                  iv0, iv1, buf0, buf1, sem0, sem1):
    wid = lax.axis_index("s") * _NC + lax.axis_index("c")
    nj = (nch + _NW - 1) // _NW

    def step(j, carry):
        c = wid + j * _NW

        @pl.when(c < nch)
        def _():
            r0 = c * _CK
            i0 = pltpu.async_copy(idx0.at[pl.ds(r0, _CK)], iv0, sem0)
            i1 = pltpu.async_copy(idx1.at[pl.ds(r0, _CK)], iv1, sem0)
            i0.wait()
            i1.wait()
            cp0 = pltpu.async_copy(table.at[iv0], buf0, sem1)
            cp1 = pltpu.async_copy(table.at[iv1], buf1, sem1)
            cp0.wait()
            cp1.wait()
            w0 = pltpu.async_copy(buf0, out0.at[pl.ds(r0, _CK)], sem0)
            w1 = pltpu.async_copy(buf1, out1.at[pl.ds(r0, _CK)], sem0)
            w0.wait()
            w1.wait()

        return carry

    lax.fori_loop(0, nj, step, 0)


def _gather2(table, idx0, idx1):
    e = idx0.shape[0]
    nch = e // _CK
    out = jax.ShapeDtypeStruct((e, _D), jnp.float32)
    f = pl.kernel(
        functools.partial(_gather2_body, nch),
        out_type=(out, out),
        mesh=_mesh(),
        scratch_types=[
            pltpu.VMEM((_CK,), jnp.int32),
            pltpu.VMEM((_CK,), jnp.int32),
            pltpu.VMEM((_CK, _D), jnp.float32),
            pltpu.VMEM((_CK, _D), jnp.float32),
            pltpu.SemaphoreType.DMA,
            pltpu.SemaphoreType.DMA,
        ],
    )
    return f(table, idx0, idx1)


def _gather5_body(nch, bond, bnw, atom, idx_s, idx_d, idx_v,
                  o_bs, o_bd, o_ws, o_wd, o_vf,
                  ivs, ivd, ivv, b0, b1, b2, b3, b4,
                  s0, s1, s2, s3, s4):
    wid = lax.axis_index("s") * _NC + lax.axis_index("c")
    nj = (nch + _NW - 1) // _NW

    def step(j, carry):
        c = wid + j * _NW

        @pl.when(c < nch)
        def _():
            r0 = c * _CK
            i0 = pltpu.async_copy(idx_s.at[pl.ds(r0, _CK)], ivs, s0)
            i1 = pltpu.async_copy(idx_d.at[pl.ds(r0, _CK)], ivd, s0)
            i2 = pltpu.async_copy(idx_v.at[pl.ds(r0, _CK)], ivv, s0)
            i0.wait()
            i1.wait()
            i2.wait()
            c0 = pltpu.async_copy(bond.at[ivs], b0, s1)
            c1 = pltpu.async_copy(bond.at[ivd], b1, s1)
            c2 = pltpu.async_copy(bnw.at[ivs], b2, s2)
            c3 = pltpu.async_copy(bnw.at[ivd], b3, s2)
            c4 = pltpu.async_copy(atom.at[ivv], b4, s3)
            c0.wait()
            c1.wait()
            c2.wait()
            c3.wait()
            c4.wait()
            w0 = pltpu.async_copy(b0, o_bs.at[pl.ds(r0, _CK)], s4)
            w1 = pltpu.async_copy(b1, o_bd.at[pl.ds(r0, _CK)], s4)
            w2 = pltpu.async_copy(b2, o_ws.at[pl.ds(r0, _CK)], s4)
            w3 = pltpu.async_copy(b3, o_wd.at[pl.ds(r0, _CK)], s4)
            w4 = pltpu.async_copy(b4, o_vf.at[pl.ds(r0, _CK)], s4)
            w0.wait()
            w1.wait()
            w2.wait()
            w3.wait()
            w4.wait()

        return carry

    lax.fori_loop(0, nj, step, 0)


def _gather5(bond, bnw, atom, idx_s, idx_d, idx_v):
    e = idx_s.shape[0]
    nch = e // _CK
    out = jax.ShapeDtypeStruct((e, _D), jnp.float32)
    f = pl.kernel(
        functools.partial(_gather5_body, nch),
        out_type=(out,) * 5,
        mesh=_mesh(),
        scratch_types=[
            pltpu.VMEM((_CK,), jnp.int32),
            pltpu.VMEM((_CK,), jnp.int32),
            pltpu.VMEM((_CK,), jnp.int32),
        ] + [pltpu.VMEM((_CK, _D), jnp.float32)] * 5
          + [pltpu.SemaphoreType.DMA] * 5,
    )
    return f(bond, bnw, atom, idx_s, idx_d, idx_v)


def _segsum1_body(m, dstv, hout, acc, buf, lidx, zb):
    cid = lax.axis_index("c")
    sid = lax.axis_index("s")
    _zero_zb(zb)
    rpt = _SP1 // _NS
    for r in range(rpt // _CK):
        pltpu.sync_copy(zb, acc.at[pl.ds(sid * rpt + r * _CK, _CK)])
    plsc.subcore_barrier()

    nch = _EA // _CK
    nch_core = nch // _NC
    nj = (nch_core + _NS - 1) // _NS

    def step(j, carry):
        lc = sid + j * _NS

        @pl.when(lc < nch_core)
        def _():
            r0 = (cid * nch_core + lc) * _CK
            pltpu.sync_copy(dstv.at[pl.ds(r0, _CK)], lidx)
            pltpu.sync_copy(m.at[pl.ds(r0, _CK)], buf)
            pltpu.sync_copy(buf, acc.at[lidx], add=True)

        return carry

    lax.fori_loop(0, nj, step, 0)
    plsc.subcore_barrier()
    for r in range(rpt // _CK):
        rr = sid * rpt + r * _CK
        pltpu.sync_copy(acc.at[pl.ds(rr, _CK)], buf)
        pltpu.sync_copy(buf, hout.at[cid, pl.ds(rr, _CK)])


def _segsum1(m, dstv):
    f = pl.kernel(
        _segsum1_body,
        out_type=jax.ShapeDtypeStruct((_NC, _SP1, _D), jnp.float32),
        mesh=_mesh(),
        scratch_types=[
            pltpu.VMEM_SHARED((_SP1, _D), jnp.float32),
            pltpu.VMEM((_CK, _D), jnp.float32),
            pltpu.VMEM((_CK,), jnp.int32),
            pltpu.VMEM((_CK, _D), jnp.float32),
        ],
    )
    return f(m, dstv)


def _segsum2_body(m, dstv, hout, acc, dslice, ids, rel, idrow, relrow,
                  mbuf, sem):
    cid = lax.axis_index("c")
    sid = lax.axis_index("s")
    ept = _EB // _NS
    pltpu.sync_copy(dstv.at[pl.ds(sid * ept, ept)], dslice)
    rpt = _SP2C // _NS
    iota16 = lax.iota(jnp.int32, 16)
    dump = jnp.full((16,), _SP2C, jnp.int32)
    zid = jnp.zeros((16,), jnp.int32)
    zf = jnp.zeros((16,), jnp.float32)
    ones16 = jnp.ones((16,), jnp.int32)
    zeros16 = jnp.zeros((16,), jnp.int32)
    fifteen16 = jnp.full((16,), 15, jnp.int32)
    ebase = sid * ept
    segs = [(0, 5120), (5120, 5120), (10240, 5120), (15360, 4640)]

    def chunk_loop(kc, carry):
        lo = (cid + kc * _NC) * _SP2C

        def zrow(i, zcarry):
            for v in range(_D // 16):
                mbuf[i, pl.ds(v * 16, 16)] = zf
            return zcarry

        lax.fori_loop(0, _CK, zrow, 0)
        for r in range(rpt // _CK):
            pltpu.sync_copy(mbuf, acc.at[pl.ds(sid * rpt + r * _CK, _CK)])
        plsc.subcore_barrier()

        for sbase, slen in segs:
            def scanv(v, cnt):
                d = dslice[pl.ds(sbase + v * 16, 16)]
                rel16 = d - lo
                ok = (rel16 >= 0) & (rel16 < _SP2C)
                oki = jnp.where(ok, ones16, zeros16)
                s = oki
                for sh in (1, 2, 4, 8):
                    s = s + jnp.where(iota16 >= sh,
                                      jnp.take(s, jnp.maximum(iota16 - sh, 0)),
                                      zeros16)
                idx = zeros16
                tgt = iota16 + 1
                for sh in (8, 4, 2, 1):
                    cand = idx + sh
                    sval = jnp.take(s, cand - 1)
                    idx = jnp.where(sval < tgt, cand, idx)
                k = s[15]
                safe = jnp.minimum(idx, fifteen16)
                crel = jnp.where(iota16 < k, jnp.take(rel16, safe), dump)
                ids[pl.ds(cnt, 16)] = (ebase + sbase + v * 16) + safe
                rel[pl.ds(cnt, 16)] = crel
                return cnt + k

            cnt = lax.fori_loop(0, slen // 16, scanv, jnp.int32(0))
            for t in range(8):
                ids[pl.ds(cnt + t * 16, 16)] = zid
                rel[pl.ds(cnt + t * 16, 16)] = dump

            def gat(j, gcarry):
                for v in range(_CK // 16):
                    idrow[pl.ds(v * 16, 16)] = ids[pl.ds(j * _CK + v * 16, 16)]
                    relrow[pl.ds(v * 16, 16)] = rel[pl.ds(j * _CK + v * 16, 16)]
                cp = pltpu.async_copy(m.at[idrow], mbuf, sem)
                cp.wait()
                pltpu.sync_copy(mbuf, acc.at[relrow], add=True)
                return gcarry

            lax.fori_loop(0, (cnt + _CK - 1) // _CK, gat, 0)

        plsc.subcore_barrier()
        for r in range(rpt // _CK):
            rr = sid * rpt + r * _CK
            pltpu.sync_copy(acc.at[pl.ds(rr, _CK)], mbuf)
            pltpu.sync_copy(mbuf, hout.at[pl.ds(lo + rr, _CK)])
        plsc.subcore_barrier()
        return carry

    lax.fori_loop(0, _NCH2 // _NC, chunk_loop, 0)


def _segsum2(m, dstv):
    ept = _EB // _NS
    f = pl.kernel(
        _segsum2_body,
        out_type=jax.ShapeDtypeStruct((_SP2, _D), jnp.float32),
        mesh=_mesh(),
        scratch_types=[
            pltpu.VMEM_SHARED((_SP2C + 8, _D), jnp.float32),
            pltpu.VMEM((ept,), jnp.int32),
            pltpu.VMEM((5248,), jnp.int32),
            pltpu.VMEM((5248,), jnp.int32),
            pltpu.VMEM((_CK,), jnp.int32),
            pltpu.VMEM((_CK,), jnp.int32),
            pltpu.VMEM((_CK, _D), jnp.float32),
            pltpu.SemaphoreType.DMA,
        ],
    )
    return f(m, dstv)


_BE = 1280


def _sigmoid(x):
    return 1.0 / (1.0 + jnp.exp(-x))


def _silu(x):
    return x * _sigmoid(x)


def _mlp_body(n_in, n_mult, residual, *refs):
    xs = refs[:n_in]
    i = n_in
    mults = refs[i:i + n_mult]
    i += n_mult
    res = refs[i] if residual else None
    i += 1 if residual else 0
    gw1, gb1, gw2, gb2, ow1, ob1, ow2, ob2, out = refs[i:i + 9]

    x0 = xs[0][...]
    ag = jnp.dot(x0, gw1[0], preferred_element_type=jnp.float32)
    ao = jnp.dot(x0, ow1[0], preferred_element_type=jnp.float32)
    for k in range(1, n_in):
        xk = xs[k][...]
        ag += jnp.dot(xk, gw1[k], preferred_element_type=jnp.float32)
        ao += jnp.dot(xk, ow1[k], preferred_element_type=jnp.float32)
    hg = _silu(ag + gb1[...])
    ho = _silu(ao + ob1[...])
    g = _sigmoid(jnp.dot(hg, gw2[...], preferred_element_type=jnp.float32)
                 + gb2[...])
    o = _silu(jnp.dot(ho, ow2[...], preferred_element_type=jnp.float32)
              + ob2[...])
    y = o * g
    for mr in mults:
        y = y * mr[...]
    if residual:
        y = y + res[...]
    out[...] = y


def _mlp(xs, mults, res, p):
    e = xs[0].shape[0]
    n_in = len(xs)
    grid = (e // _BE,)
    row = pl.BlockSpec((_BE, _D), lambda i: (i, 0))
    w1s = pl.BlockSpec((n_in, _D, _H), lambda i: (0, 0, 0))
    b1s = pl.BlockSpec((1, _H), lambda i: (0, 0))
    w2s = pl.BlockSpec((_H, _D), lambda i: (0, 0))
    b2s = pl.BlockSpec((1, _D), lambda i: (0, 0))
    n_row = n_in + len(mults) + (1 if res is not None else 0)
    in_specs = [row] * n_row + [w1s, b1s, w2s, b2s, w1s, b1s, w2s, b2s]
    gw1 = p['gw1'].reshape(n_in, _D, _H)
    ow1 = p['ow1'].reshape(n_in, _D, _H)
    args = ([*xs, *mults] + ([res] if res is not None else [])
            + [gw1, p['gb1'].reshape(1, _H), p['gw2'], p['gb2'].reshape(1, _D),
               ow1, p['ob1'].reshape(1, _H), p['ow2'], p['ob2'].reshape(1, _D)])
    return pl.pallas_call(
        functools.partial(_mlp_body, n_in, len(mults), res is not None),
        grid=grid,
        in_specs=in_specs,
        out_specs=row,
        out_shape=jax.ShapeDtypeStruct((e, _D), jnp.float32),
        compiler_params=pltpu.CompilerParams(
            dimension_semantics=("arbitrary",)),
    )(*args)


def _lin_body(nh, *refs):
    feat = refs[0]
    hs = refs[1:1 + nh]
    w, b, out = refs[1 + nh:1 + nh + 3]
    h = hs[0][...]
    for k in range(1, nh):
        h = h + hs[k][...]
    out[...] = (feat[...]
                + jnp.dot(h, w[...], preferred_element_type=jnp.float32)
                + b[...])


def _lin(feat, hs, w, b, be):
    e = feat.shape[0]
    grid = (e // be,)
    row = pl.BlockSpec((be, _D), lambda i: (i, 0))
    ws = pl.BlockSpec((_D, _D), lambda i: (0, 0))
    bs = pl.BlockSpec((1, _D), lambda i: (0, 0))
    return pl.pallas_call(
        functools.partial(_lin_body, len(hs)),
        grid=grid,
        in_specs=[row] * (1 + len(hs)) + [ws, bs],
        out_specs=row,
        out_shape=jax.ShapeDtypeStruct((e, _D), jnp.float32),
        compiler_params=pltpu.CompilerParams(
            dimension_semantics=("arbitrary",)),
    )(feat, *hs, w, b.reshape(1, _D))


def kernel(atom_feat, bond_feat, angle_feat, atom_edge_index, bond_edge_index,
           angle_index, atom_bond_weight, bond_node_weight, params):
    src_a = atom_edge_index[0]
    dst_a = atom_edge_index[1]
    src_b = bond_edge_index[0]
    dst_b = bond_edge_index[1]
    vertex = angle_index[:, 1]

    g1s, g1d = _gather2(atom_feat, src_a, dst_a)
    m1 = _mlp([g1s, g1d, bond_feat], [atom_bond_weight], None,
              params['atom_conv'])
    hparts = _segsum1(m1, dst_a)
    atom_out = _lin(atom_feat, [hparts[0, :_NA], hparts[1, :_NA]],
                    params['atom_lin']['w'], params['atom_lin']['b'], 1000)

    bfs, bfd, ws, wd, vf = _gather5(bond_feat, bond_node_weight, atom_out,
                                    src_b, dst_b, vertex)
    m2 = _mlp([bfs, bfd, angle_feat, vf], [ws, wd], None, params['bond_conv'])
    h2 = _segsum2(m2, dst_b)
    bond_out = _lin(bond_feat, [h2[:_EA]],
                    params['bond_lin']['w'], params['bond_lin']['b'], _BE)

    g3s, g3d = _gather2(bond_out, src_b, dst_b)
    angle_out = _mlp([g3s, g3d, angle_feat, vf], [], angle_feat,
                     params['angle_update'])

    return (atom_out, bond_out, angle_out)
